# baseline (speedup 1.0000x reference)
import functools

import jax
import jax.numpy as jnp
from jax import lax
from jax.experimental import pallas as pl
from jax.experimental.pallas import tpu as pltpu
from jax.experimental.pallas import tpu_sc as plsc

NUM_TYPES = 22
HIDDEN_DIM = 256
N_TOKENS = 16384

_NC = 2
_NS = 16
_NW = _NC * _NS
_ROWS_PER_W = N_TOKENS // _NW
_CHUNK = 64
_NCHUNKS = _ROWS_PER_W // _CHUNK
_NBUF = 3
_L = 16
_VPR = HIDDEN_DIM // _L
_GROUPS = _CHUNK // _L


def _make_sc_gather():
  mesh = plsc.VectorSubcoreMesh(core_axis_name="c", subcore_axis_name="s")

  @functools.partial(
      pl.kernel,
      mesh=mesh,
      compiler_params=pltpu.CompilerParams(needs_layout_passes=False),
      out_type=jax.ShapeDtypeStruct((N_TOKENS, HIDDEN_DIM), jnp.float32),
      scratch_types=(
          [pltpu.VMEM((_ROWS_PER_W,), jnp.int32),
           pltpu.VMEM((NUM_TYPES, HIDDEN_DIM), jnp.float32)]
          + [pltpu.VMEM((_CHUNK, HIDDEN_DIM), jnp.float32)] * _NBUF
          + [pltpu.SemaphoreType.DMA] * _NBUF
      ),
  )
  def k(idx_hbm, table_hbm, out_hbm, idx_v, table_v, *rest):
    bufs = rest[:_NBUF]
    ssem = rest[_NBUF:]
    wid = lax.axis_index("s") * _NC + lax.axis_index("c")
    base = wid * _ROWS_PER_W
    pltpu.sync_copy(table_hbm, table_v)
    pltpu.sync_copy(idx_hbm.at[pl.ds(base, _ROWS_PER_W)], idx_v)

    def fill_chunk(j, buf):
      @plsc.parallel_loop(0, _GROUPS, unroll=2)
      def group(g):
        t16 = idx_v[pl.ds(j * _CHUNK + g * _L, _L)]
        prev = None
        for r in range(_L):
          t = t16[r]
          vals = [table_v[t, pl.ds(c * _L, _L)] for c in range(_VPR)]
          if prev is not None:
            prow, pvals = prev
            for c in range(_VPR):
              buf[prow, pl.ds(c * _L, _L)] = pvals[c]
          prev = (g * _L + r, vals)
        prow, pvals = prev
        for c in range(_VPR):
          buf[prow, pl.ds(c * _L, _L)] = pvals[c]

    stores = [None] * _NCHUNKS
    for j in range(_NCHUNKS):
      b = j % _NBUF
      if j >= _NBUF:
        stores[j - _NBUF].wait()
      fill_chunk(j, bufs[b])
      stores[j] = pltpu.async_copy(
          bufs[b], out_hbm.at[pl.ds(base + j * _CHUNK, _CHUNK)], ssem[b])
    for j in range(max(0, _NCHUNKS - _NBUF), _NCHUNKS):
      stores[j].wait()

  return k


_sc_gather = _make_sc_gather()


def kernel(deriv_types, embedding_weight):
  idx = deriv_types.astype(jnp.int32)
  return _sc_gather(idx, embedding_weight)


# speedup vs baseline: 1.1651x; 1.1651x over previous
import functools

import jax
import jax.numpy as jnp
from jax import lax
from jax.experimental import pallas as pl
from jax.experimental.pallas import tpu as pltpu
from jax.experimental.pallas import tpu_sc as plsc

NUM_TYPES = 22
HIDDEN_DIM = 256
N_TOKENS = 16384

_NC = 2
_NS = 16
_NW = _NC * _NS
_ROWS_PER_W = N_TOKENS // _NW
_CHUNK = 128
_NCHUNKS = _ROWS_PER_W // _CHUNK
_NBUF = 3
_L = 16
_VPR = HIDDEN_DIM // _L
_GROUPS = _CHUNK // _L


def _make_sc_gather():
  mesh = plsc.VectorSubcoreMesh(core_axis_name="c", subcore_axis_name="s")

  @functools.partial(
      pl.kernel,
      mesh=mesh,
      compiler_params=pltpu.CompilerParams(needs_layout_passes=False),
      out_type=jax.ShapeDtypeStruct((N_TOKENS, HIDDEN_DIM), jnp.float32),
      scratch_types=(
          [pltpu.VMEM((_ROWS_PER_W,), jnp.int32),
           pltpu.VMEM((NUM_TYPES, HIDDEN_DIM), jnp.float32)]
          + [pltpu.VMEM((_CHUNK, HIDDEN_DIM), jnp.float32)] * _NBUF
          + [pltpu.SemaphoreType.DMA] * _NBUF
      ),
  )
  def k(idx_hbm, table_hbm, out_hbm, idx_v, table_v, *rest):
    bufs = rest[:_NBUF]
    ssem = rest[_NBUF:]
    wid = lax.axis_index("s") * _NC + lax.axis_index("c")
    base = wid * _ROWS_PER_W
    pltpu.sync_copy(table_hbm, table_v)
    pltpu.sync_copy(idx_hbm.at[pl.ds(base, _ROWS_PER_W)], idx_v)

    def fill_chunk(j, buf):
      @plsc.parallel_loop(0, _GROUPS, unroll=2)
      def group(g):
        t16 = idx_v[pl.ds(j * _CHUNK + g * _L, _L)]
        prev = None
        for r in range(_L):
          t = t16[r]
          vals = [table_v[t, pl.ds(c * _L, _L)] for c in range(_VPR)]
          if prev is not None:
            prow, pvals = prev
            for c in range(_VPR):
              buf[prow, pl.ds(c * _L, _L)] = pvals[c]
          prev = (g * _L + r, vals)
        prow, pvals = prev
        for c in range(_VPR):
          buf[prow, pl.ds(c * _L, _L)] = pvals[c]

    stores = [None] * _NCHUNKS
    for j in range(_NCHUNKS):
      b = j % _NBUF
      if j >= _NBUF:
        stores[j - _NBUF].wait()
      fill_chunk(j, bufs[b])
      stores[j] = pltpu.async_copy(
          bufs[b], out_hbm.at[pl.ds(base + j * _CHUNK, _CHUNK)], ssem[b])
    for j in range(max(0, _NCHUNKS - _NBUF), _NCHUNKS):
      stores[j].wait()

  return k


_sc_gather = _make_sc_gather()


def kernel(deriv_types, embedding_weight):
  idx = deriv_types.astype(jnp.int32)
  return _sc_gather(idx, embedding_weight)


# speedup vs baseline: 1.3080x; 1.1227x over previous
import functools

import jax
import jax.numpy as jnp
from jax import lax
from jax.experimental import pallas as pl
from jax.experimental.pallas import tpu as pltpu
from jax.experimental.pallas import tpu_sc as plsc

NUM_TYPES = 22
HIDDEN_DIM = 256
N_TOKENS = 16384

_NC = 2
_NS = 16
_NW = _NC * _NS
_ROWS_PER_W = N_TOKENS // _NW
_CHUNK = 128
_NCHUNKS = _ROWS_PER_W // _CHUNK
_NBUF = 3
_L = 16
_VPR = HIDDEN_DIM // _L
_GROUPS = _CHUNK // _L


def _make_sc_gather():
  mesh = plsc.VectorSubcoreMesh(core_axis_name="c", subcore_axis_name="s")

  @functools.partial(
      pl.kernel,
      mesh=mesh,
      compiler_params=pltpu.CompilerParams(needs_layout_passes=False),
      out_type=jax.ShapeDtypeStruct((N_TOKENS, HIDDEN_DIM), jnp.float32),
      scratch_types=(
          [pltpu.VMEM((_ROWS_PER_W,), jnp.int32),
           pltpu.VMEM((NUM_TYPES, HIDDEN_DIM), jnp.float32)]
          + [pltpu.VMEM((_CHUNK, HIDDEN_DIM), jnp.float32)] * _NBUF
          + [pltpu.SemaphoreType.DMA] * _NBUF
      ),
  )
  def k(idx_hbm, table_hbm, out_hbm, idx_v, table_v, *rest):
    bufs = rest[:_NBUF]
    ssem = rest[_NBUF:]
    wid = lax.axis_index("s") * _NC + lax.axis_index("c")
    base = wid * _ROWS_PER_W
    pltpu.sync_copy(table_hbm, table_v)
    pltpu.sync_copy(idx_hbm.at[pl.ds(base, _ROWS_PER_W)], idx_v)

    def fill_chunk(j, buf):
      @plsc.parallel_loop(0, _GROUPS)
      def group(g):
        t16 = idx_v[pl.ds(j * _CHUNK + g * _L, _L)]
        prev = None
        for r in range(_L):
          t = t16[r]
          vals = [table_v[t, pl.ds(c * _L, _L)] for c in range(_VPR)]
          if prev is not None:
            prow, pvals = prev
            for c in range(_VPR):
              buf[prow, pl.ds(c * _L, _L)] = pvals[c]
          prev = (g * _L + r, vals)
        prow, pvals = prev
        for c in range(_VPR):
          buf[prow, pl.ds(c * _L, _L)] = pvals[c]

    stores = [None] * _NCHUNKS
    for j in range(_NCHUNKS):
      b = j % _NBUF
      if j >= _NBUF:
        stores[j - _NBUF].wait()
      fill_chunk(j, bufs[b])
      stores[j] = pltpu.async_copy(
          bufs[b], out_hbm.at[pl.ds(base + j * _CHUNK, _CHUNK)], ssem[b])
    for j in range(max(0, _NCHUNKS - _NBUF), _NCHUNKS):
      stores[j].wait()

  return k


_sc_gather = _make_sc_gather()


def kernel(deriv_types, embedding_weight):
  idx = deriv_types.astype(jnp.int32)
  return _sc_gather(idx, embedding_weight)
